# Initial kernel scaffold; baseline (speedup 1.0000x reference)
import jax
import jax.numpy as jnp
from jax.experimental import pallas as pl


def kernel(style_ids, key_ids, section_ids, groove_features, style_table, key_table, section_table, groove_W, groove_b):
    raise NotImplementedError("write your pallas kernel here")


# speedup vs baseline: 8.0373x; 8.0373x over previous
import jax
import jax.numpy as jnp
from jax.experimental import pallas as pl

_B = 16384
_D = 128
_R = 2048


def _body(ids_ref, g_ref, w_ref, t_ref, b_ref, o_ref):
    ids = ids_ref[0]
    cols = jax.lax.broadcasted_iota(jnp.int32, (_R, 32), 1)
    oh = (
        (cols == ids[0][:, None])
        | (cols == ids[1][:, None])
        | (cols == ids[2][:, None])
    ).astype(jnp.float32)
    acc = jnp.dot(g_ref[...], w_ref[...], preferred_element_type=jnp.float32)
    acc += jnp.dot(oh, t_ref[...], preferred_element_type=jnp.float32)
    o_ref[...] = acc + b_ref[...]


def kernel(style_ids, key_ids, section_ids, groove_features, style_table,
           key_table, section_table, groove_W, groove_b):
    nb = _B // _R
    ids3 = jnp.stack(
        [style_ids.astype(jnp.int32),
         key_ids.astype(jnp.int32) + 3,
         section_ids.astype(jnp.int32) + 27],
        axis=0,
    )
    ids3 = ids3.reshape(3, nb, _R).transpose(1, 0, 2)
    tables = jnp.concatenate([style_table, key_table, section_table], axis=0)
    bias = groove_b.reshape(1, _D)

    return pl.pallas_call(
        _body,
        grid=(nb,),
        in_specs=[
            pl.BlockSpec((1, 3, _R), lambda i: (i, 0, 0)),
            pl.BlockSpec((_R, 32), lambda i: (i, 0)),
            pl.BlockSpec((32, _D), lambda i: (0, 0)),
            pl.BlockSpec((32, _D), lambda i: (0, 0)),
            pl.BlockSpec((1, _D), lambda i: (0, 0)),
        ],
        out_specs=pl.BlockSpec((_R, _D), lambda i: (i, 0)),
        out_shape=jax.ShapeDtypeStruct((_B, _D), jnp.float32),
    )(ids3, groove_features, groove_W, tables, bias)


# speedup vs baseline: 8.7070x; 1.0833x over previous
import jax
import jax.numpy as jnp
from jax.experimental import pallas as pl

_B = 16384
_D = 128
_R = 4096


def _body(ids_ref, g_ref, w_ref, t_ref, b_ref, o_ref):
    ids = ids_ref[0]
    cols = jax.lax.broadcasted_iota(jnp.int32, (_R, 32), 1)
    oh = (
        (cols == ids[0][:, None])
        | (cols == ids[1][:, None])
        | (cols == ids[2][:, None])
    ).astype(jnp.float32)
    acc = jnp.dot(g_ref[...], w_ref[...], preferred_element_type=jnp.float32)
    acc += jnp.dot(oh, t_ref[...], preferred_element_type=jnp.float32)
    o_ref[...] = acc + b_ref[...]


def kernel(style_ids, key_ids, section_ids, groove_features, style_table,
           key_table, section_table, groove_W, groove_b):
    nb = _B // _R
    ids3 = jnp.stack(
        [style_ids.astype(jnp.int32),
         key_ids.astype(jnp.int32) + 3,
         section_ids.astype(jnp.int32) + 27],
        axis=0,
    )
    ids3 = ids3.reshape(3, nb, _R).transpose(1, 0, 2)
    tables = jnp.concatenate([style_table, key_table, section_table], axis=0)
    bias = groove_b.reshape(1, _D)

    return pl.pallas_call(
        _body,
        grid=(nb,),
        in_specs=[
            pl.BlockSpec((1, 3, _R), lambda i: (i, 0, 0)),
            pl.BlockSpec((_R, 32), lambda i: (i, 0)),
            pl.BlockSpec((32, _D), lambda i: (0, 0)),
            pl.BlockSpec((32, _D), lambda i: (0, 0)),
            pl.BlockSpec((1, _D), lambda i: (0, 0)),
        ],
        out_specs=pl.BlockSpec((_R, _D), lambda i: (i, 0)),
        out_shape=jax.ShapeDtypeStruct((_B, _D), jnp.float32),
    )(ids3, groove_features, groove_W, tables, bias)
